# Initial kernel scaffold; baseline (speedup 1.0000x reference)
import jax
import jax.numpy as jnp
from jax.experimental import pallas as pl


def kernel(pos_u, pos_v, neg_v, aux_pos_u, aux_pos_v, aux_neg_v, u_emb, v_emb, aux_u_emb, aux_v_emb):
    raise NotImplementedError("write your pallas kernel here")


# speedup vs baseline: 1.7419x; 1.7419x over previous
import functools

import jax
import jax.numpy as jnp
from jax import lax
from jax.experimental import pallas as pl
from jax.experimental.pallas import tpu as pltpu
from jax.experimental.pallas import tpu_sc as plsc

VOCAB = 1000000
AUX_VOCAB = 100000
DIM = 64
AUX_DIM = 32
B = 16384
NNEG = 5
EPS = 1e-05

NC = 2
NS = 16
L = 16
NW = NC * NS
S_PER_W = B // NW
C = 128
NCHUNK = S_PER_W // C
NG = C // L
ND = DIM // L
NAD = AUX_DIM // L


def _lane_sum(p, rots):
    for r in rots:
        p = p + jnp.take(p, r)
    return p


def _sc_body(pos_u, pos_v, neg_v, apos_u, apos_v, aneg_v,
             u_emb, v_emb, au_emb, av_emb,
             pos_out, aux_out, neg_out, auxneg_out,
             iu, iv, ineg, iau, iav, ianeg,
             ur, vr, nr, aur, avr, anr,
             dots_pos, dots_aux, dots_neg, dots_auxneg,
             sem):
    wid = lax.axis_index("s") * NC + lax.axis_index("c")
    base = wid * S_PER_W
    lane = lax.iota(jnp.int32, L)
    rots = [(lane + sh) % L for sh in (8, 4, 2, 1)]
    zero = jnp.zeros((L,), jnp.float32)

    def chunk(ck, _):
        off = base + ck * C

        pltpu.sync_copy(pos_u.at[pl.ds(off, C)], iu)
        pltpu.sync_copy(pos_v.at[pl.ds(off, C)], iv)
        pltpu.sync_copy(neg_v.at[pl.ds(off * NNEG, C * NNEG)], ineg)
        pltpu.sync_copy(apos_u.at[pl.ds(off, C)], iau)
        pltpu.sync_copy(apos_v.at[pl.ds(off, C)], iav)
        pltpu.sync_copy(aneg_v.at[pl.ds(off * NNEG, C * NNEG)], ianeg)

        cps = [pltpu.async_copy(u_emb.at[iu], ur, sem),
               pltpu.async_copy(v_emb.at[iv], vr, sem),
               pltpu.async_copy(au_emb.at[iau], aur, sem),
               pltpu.async_copy(av_emb.at[iav], avr, sem)]
        for r in range(NNEG):
            cps.append(pltpu.async_copy(
                v_emb.at[ineg.at[pl.ds(r * C, C)]],
                nr.at[pl.ds(r * C, C)], sem))
            cps.append(pltpu.async_copy(
                av_emb.at[ianeg.at[pl.ds(r * C, C)]],
                anr.at[pl.ds(r * C, C)], sem))
        for cp in cps:
            cp.wait()

        def group(g, _):
            def samp(i, accs):
                apos, aaux, aneg, aaneg = accs
                s = g * L + i
                sel = lane == i
                urow = ur.at[s]
                vrow = vr.at[s]
                us = [urow[pl.ds(k * L, L)] for k in range(ND)]
                p = us[0] * vrow[pl.ds(0, L)]
                for k in range(1, ND):
                    p = p + us[k] * vrow[pl.ds(k * L, L)]
                apos = jnp.where(sel, _lane_sum(p, rots), apos)

                aurow = aur.at[s]
                avrow = avr.at[s]
                aus = [aurow[pl.ds(k * L, L)] for k in range(NAD)]
                a = aus[0] * avrow[pl.ds(0, L)]
                for k in range(1, NAD):
                    a = a + aus[k] * avrow[pl.ds(k * L, L)]
                aaux = jnp.where(sel, _lane_sum(a, rots), aaux)

                aneg2, aaneg2 = [], []
                for n in range(NNEG):
                    nrow = nr.at[n * C + s]
                    q = us[0] * nrow[pl.ds(0, L)]
                    for k in range(1, ND):
                        q = q + us[k] * nrow[pl.ds(k * L, L)]
                    aneg2.append(jnp.where(sel, _lane_sum(q, rots), aneg[n]))

                    anrow = anr.at[n * C + s]
                    aq = aus[0] * anrow[pl.ds(0, L)]
                    for k in range(1, NAD):
                        aq = aq + aus[k] * anrow[pl.ds(k * L, L)]
                    aaneg2.append(
                        jnp.where(sel, _lane_sum(aq, rots), aaneg[n]))
                return apos, aaux, tuple(aneg2), tuple(aaneg2)

            init = (zero, zero, (zero,) * NNEG, (zero,) * NNEG)
            apos, aaux, aneg, aaneg = lax.fori_loop(0, L, samp, init)

            dots_pos[pl.ds(g * L, L)] = apos
            dots_aux[pl.ds(g * L, L)] = aaux
            for n in range(NNEG):
                dots_neg[pl.ds(n * C + g * L, L)] = aneg[n]
                dots_auxneg[pl.ds(n * C + g * L, L)] = aaneg[n]
            return 0

        lax.fori_loop(0, NG, group, 0)

        pltpu.sync_copy(dots_pos, pos_out.at[pl.ds(off, C)])
        pltpu.sync_copy(dots_aux, aux_out.at[pl.ds(off, C)])
        for n in range(NNEG):
            pltpu.sync_copy(dots_neg.at[pl.ds(n * C, C)],
                            neg_out.at[pl.ds(n * B + off, C)])
            pltpu.sync_copy(dots_auxneg.at[pl.ds(n * C, C)],
                            auxneg_out.at[pl.ds(n * B + off, C)])
        return 0

    lax.fori_loop(0, NCHUNK, chunk, 0)


@functools.cache
def _make_sc_dots():
  return functools.partial(
    pl.kernel,
    out_type=[
        jax.ShapeDtypeStruct((B,), jnp.float32),
        jax.ShapeDtypeStruct((B,), jnp.float32),
        jax.ShapeDtypeStruct((NNEG * B,), jnp.float32),
        jax.ShapeDtypeStruct((NNEG * B,), jnp.float32),
    ],
    mesh=plsc.VectorSubcoreMesh(core_axis_name="c", subcore_axis_name="s",
                                num_cores=NC, num_subcores=NS),
    scratch_types=[
        pltpu.VMEM((C,), jnp.int32),
        pltpu.VMEM((C,), jnp.int32),
        pltpu.VMEM((C * NNEG,), jnp.int32),
        pltpu.VMEM((C,), jnp.int32),
        pltpu.VMEM((C,), jnp.int32),
        pltpu.VMEM((C * NNEG,), jnp.int32),
        pltpu.VMEM((C, DIM), jnp.float32),
        pltpu.VMEM((C, DIM), jnp.float32),
        pltpu.VMEM((C * NNEG, DIM), jnp.float32),
        pltpu.VMEM((C, AUX_DIM), jnp.float32),
        pltpu.VMEM((C, AUX_DIM), jnp.float32),
        pltpu.VMEM((C * NNEG, AUX_DIM), jnp.float32),
        pltpu.VMEM((C,), jnp.float32),
        pltpu.VMEM((C,), jnp.float32),
        pltpu.VMEM((C * NNEG,), jnp.float32),
        pltpu.VMEM((C * NNEG,), jnp.float32),
        pltpu.SemaphoreType.DMA,
    ],
    compiler_params=pltpu.CompilerParams(use_tc_tiling_on_sc=False),
  )(_sc_body)


def _tc_body(pos_ref, aux_ref, neg_ref, aneg_ref, o1_ref, o2_ref):
    pos = pos_ref[...]
    f1 = -jnp.log(jnp.clip(pos, EPS, 1.0 - EPS))
    neg = neg_ref[...]
    g1 = jnp.log(1.0 - jnp.clip(neg, EPS, 1.0 - EPS))
    o1_ref[...] = jnp.reshape((jnp.sum(f1) - jnp.sum(g1)) / B, (1, 1))

    aux = aux_ref[...]
    f2 = -jnp.log(jnp.clip(aux, EPS, 1.0 - EPS))
    an = aneg_ref[...]
    g2 = jnp.log(1.0 - jnp.clip(an, EPS, 1.0 - EPS))
    o2_ref[...] = jnp.reshape((jnp.sum(f2) - jnp.sum(g2)) / B, (1, 1))


def _tc_loss(pos_d, aux_d, neg_d, aneg_d):
    return pl.pallas_call(
        _tc_body,
        out_shape=[jax.ShapeDtypeStruct((1, 1), jnp.float32),
                   jax.ShapeDtypeStruct((1, 1), jnp.float32)],
    )(pos_d, aux_d, neg_d, aneg_d)


def kernel(pos_u, pos_v, neg_v, aux_pos_u, aux_pos_v, aux_neg_v,
           u_emb, v_emb, aux_u_emb, aux_v_emb):
    pos_u = pos_u.astype(jnp.int32)
    pos_v = pos_v.astype(jnp.int32)
    neg_flat = neg_v.reshape(-1).astype(jnp.int32)
    aux_pos_u = aux_pos_u.astype(jnp.int32)
    aux_pos_v = aux_pos_v.astype(jnp.int32)
    aneg_flat = aux_neg_v.reshape(-1).astype(jnp.int32)

    pos_d, aux_d, neg_d, aneg_d = _make_sc_dots()(
        pos_u, pos_v, neg_flat, aux_pos_u, aux_pos_v, aneg_flat,
        u_emb, v_emb, aux_u_emb, aux_v_emb)

    o1, o2 = _tc_loss(pos_d.reshape(B // 128, 128),
                      aux_d.reshape(B // 128, 128),
                      neg_d.reshape(NNEG * B // 128, 128),
                      aneg_d.reshape(NNEG * B // 128, 128))
    return (o1[0, 0], o2[0, 0])


# speedup vs baseline: 1.9416x; 1.1146x over previous
import functools

import jax
import jax.numpy as jnp
from jax import lax
from jax.experimental import pallas as pl
from jax.experimental.pallas import tpu as pltpu
from jax.experimental.pallas import tpu_sc as plsc

VOCAB = 1000000
AUX_VOCAB = 100000
DIM = 64
AUX_DIM = 32
B = 16384
NNEG = 5
EPS = 1e-05

NC = 2
NS = 16
L = 16
NW = NC * NS
S_PER_W = B // NW
C = 64
NCHUNK = S_PER_W // C
NG = C // L
ND = DIM // L
NAD = AUX_DIM // L


def _lane_sum(p, rots):
    for r in rots:
        p = p + jnp.take(p, r)
    return p


def _sc_body(pos_u, pos_v, neg_v, apos_u, apos_v, aneg_v,
             uv_tab, aux_tab,
             pos_out, aux_out, neg_out, auxneg_out,
             iu, iv, ineg, iau, iav, ianeg,
             ur, vr, nr, aur, avr, anr,
             dots_pos, dots_aux, dots_neg, dots_auxneg,
             sem):
    wid = lax.axis_index("s") * NC + lax.axis_index("c")
    base = wid * S_PER_W
    lane = lax.iota(jnp.int32, L)
    rots = [(lane + sh) % L for sh in (8, 4, 2, 1)]
    zero = jnp.zeros((L,), jnp.float32)

    def chunk(ck, _):
        off = base + ck * C

        pltpu.sync_copy(pos_u.at[pl.ds(off, C)], iu)
        pltpu.sync_copy(pos_v.at[pl.ds(off, C)], iv)
        pltpu.sync_copy(neg_v.at[pl.ds(off * NNEG, C * NNEG)], ineg)
        pltpu.sync_copy(apos_u.at[pl.ds(off, C)], iau)
        pltpu.sync_copy(apos_v.at[pl.ds(off, C)], iav)
        pltpu.sync_copy(aneg_v.at[pl.ds(off * NNEG, C * NNEG)], ianeg)

        cps = [pltpu.async_copy(uv_tab.at[iu], ur, sem),
               pltpu.async_copy(uv_tab.at[iv], vr, sem),
               pltpu.async_copy(aux_tab.at[iau], aur, sem),
               pltpu.async_copy(aux_tab.at[iav], avr, sem)]
        for r in range(NNEG):
            cps.append(pltpu.async_copy(
                uv_tab.at[ineg.at[pl.ds(r * C, C)]],
                nr.at[pl.ds(r * C, C)], sem))
            cps.append(pltpu.async_copy(
                aux_tab.at[ianeg.at[pl.ds(r * C, C)]],
                anr.at[pl.ds(r * C, C)], sem))
        for cp in cps:
            cp.wait()

        def group(g, _):
            def samp(i, accs):
                apos, aaux, aneg, aaneg = accs
                s = g * L + i
                sel = lane == i
                urow = ur.at[s]
                vrow = vr.at[s]
                us = [urow[pl.ds(k * L, L)] for k in range(ND)]
                p = us[0] * vrow[pl.ds(DIM, L)]
                for k in range(1, ND):
                    p = p + us[k] * vrow[pl.ds(DIM + k * L, L)]
                apos = jnp.where(sel, _lane_sum(p, rots), apos)

                aurow = aur.at[s]
                avrow = avr.at[s]
                aus = [aurow[pl.ds(k * L, L)] for k in range(NAD)]
                a = aus[0] * avrow[pl.ds(AUX_DIM, L)]
                for k in range(1, NAD):
                    a = a + aus[k] * avrow[pl.ds(AUX_DIM + k * L, L)]
                aaux = jnp.where(sel, _lane_sum(a, rots), aaux)

                aneg2, aaneg2 = [], []
                for n in range(NNEG):
                    nrow = nr.at[n * C + s]
                    q = us[0] * nrow[pl.ds(DIM, L)]
                    for k in range(1, ND):
                        q = q + us[k] * nrow[pl.ds(DIM + k * L, L)]
                    aneg2.append(jnp.where(sel, _lane_sum(q, rots), aneg[n]))

                    anrow = anr.at[n * C + s]
                    aq = aus[0] * anrow[pl.ds(AUX_DIM, L)]
                    for k in range(1, NAD):
                        aq = aq + aus[k] * anrow[pl.ds(AUX_DIM + k * L, L)]
                    aaneg2.append(
                        jnp.where(sel, _lane_sum(aq, rots), aaneg[n]))
                return apos, aaux, tuple(aneg2), tuple(aaneg2)

            init = (zero, zero, (zero,) * NNEG, (zero,) * NNEG)
            apos, aaux, aneg, aaneg = lax.fori_loop(0, L, samp, init)

            dots_pos[pl.ds(g * L, L)] = apos
            dots_aux[pl.ds(g * L, L)] = aaux
            for n in range(NNEG):
                dots_neg[pl.ds(n * C + g * L, L)] = aneg[n]
                dots_auxneg[pl.ds(n * C + g * L, L)] = aaneg[n]
            return 0

        lax.fori_loop(0, NG, group, 0)

        pltpu.sync_copy(dots_pos, pos_out.at[pl.ds(off, C)])
        pltpu.sync_copy(dots_aux, aux_out.at[pl.ds(off, C)])
        for n in range(NNEG):
            pltpu.sync_copy(dots_neg.at[pl.ds(n * C, C)],
                            neg_out.at[pl.ds(n * B + off, C)])
            pltpu.sync_copy(dots_auxneg.at[pl.ds(n * C, C)],
                            auxneg_out.at[pl.ds(n * B + off, C)])
        return 0

    lax.fori_loop(0, NCHUNK, chunk, 0)


@functools.cache
def _make_sc_dots():
  return functools.partial(
    pl.kernel,
    out_type=[
        jax.ShapeDtypeStruct((B,), jnp.float32),
        jax.ShapeDtypeStruct((B,), jnp.float32),
        jax.ShapeDtypeStruct((NNEG * B,), jnp.float32),
        jax.ShapeDtypeStruct((NNEG * B,), jnp.float32),
    ],
    mesh=plsc.VectorSubcoreMesh(core_axis_name="c", subcore_axis_name="s",
                                num_cores=NC, num_subcores=NS),
    scratch_types=[
        pltpu.VMEM((C,), jnp.int32),
        pltpu.VMEM((C,), jnp.int32),
        pltpu.VMEM((C * NNEG,), jnp.int32),
        pltpu.VMEM((C,), jnp.int32),
        pltpu.VMEM((C,), jnp.int32),
        pltpu.VMEM((C * NNEG,), jnp.int32),
        pltpu.VMEM((C, 2 * DIM), jnp.float32),
        pltpu.VMEM((C, 2 * DIM), jnp.float32),
        pltpu.VMEM((C * NNEG, 2 * DIM), jnp.float32),
        pltpu.VMEM((C, 4 * AUX_DIM), jnp.float32),
        pltpu.VMEM((C, 4 * AUX_DIM), jnp.float32),
        pltpu.VMEM((C * NNEG, 4 * AUX_DIM), jnp.float32),
        pltpu.VMEM((C,), jnp.float32),
        pltpu.VMEM((C,), jnp.float32),
        pltpu.VMEM((C * NNEG,), jnp.float32),
        pltpu.VMEM((C * NNEG,), jnp.float32),
        pltpu.SemaphoreType.DMA,
    ],
  )(_sc_body)


def _tc_body(pos_ref, aux_ref, neg_ref, aneg_ref, o1_ref, o2_ref):
    pos = pos_ref[...]
    f1 = -jnp.log(jnp.clip(pos, EPS, 1.0 - EPS))
    neg = neg_ref[...]
    g1 = jnp.log(1.0 - jnp.clip(neg, EPS, 1.0 - EPS))
    o1_ref[...] = jnp.reshape((jnp.sum(f1) - jnp.sum(g1)) / B, (1, 1))

    aux = aux_ref[...]
    f2 = -jnp.log(jnp.clip(aux, EPS, 1.0 - EPS))
    an = aneg_ref[...]
    g2 = jnp.log(1.0 - jnp.clip(an, EPS, 1.0 - EPS))
    o2_ref[...] = jnp.reshape((jnp.sum(f2) - jnp.sum(g2)) / B, (1, 1))


def _tc_loss(pos_d, aux_d, neg_d, aneg_d):
    return pl.pallas_call(
        _tc_body,
        out_shape=[jax.ShapeDtypeStruct((1, 1), jnp.float32),
                   jax.ShapeDtypeStruct((1, 1), jnp.float32)],
    )(pos_d, aux_d, neg_d, aneg_d)


def kernel(pos_u, pos_v, neg_v, aux_pos_u, aux_pos_v, aux_neg_v,
           u_emb, v_emb, aux_u_emb, aux_v_emb):
    pos_u = pos_u.astype(jnp.int32)
    pos_v = pos_v.astype(jnp.int32)
    neg_flat = neg_v.reshape(-1).astype(jnp.int32)
    aux_pos_u = aux_pos_u.astype(jnp.int32)
    aux_pos_v = aux_pos_v.astype(jnp.int32)
    aneg_flat = aux_neg_v.reshape(-1).astype(jnp.int32)

    uv_tab = jnp.concatenate([u_emb, v_emb], axis=1)
    aux_tab = jnp.concatenate(
        [aux_u_emb, aux_v_emb, aux_u_emb, aux_v_emb], axis=1)

    pos_d, aux_d, neg_d, aneg_d = _make_sc_dots()(
        pos_u, pos_v, neg_flat, aux_pos_u, aux_pos_v, aneg_flat,
        uv_tab, aux_tab)

    o1, o2 = _tc_loss(pos_d.reshape(B // 128, 128),
                      aux_d.reshape(B // 128, 128),
                      neg_d.reshape(NNEG * B // 128, 128),
                      aneg_d.reshape(NNEG * B // 128, 128))
    return (o1[0, 0], o2[0, 0])


# speedup vs baseline: 2.9500x; 1.5194x over previous
import functools

import jax
import jax.numpy as jnp
from jax import lax
from jax.experimental import pallas as pl
from jax.experimental.pallas import tpu as pltpu
from jax.experimental.pallas import tpu_sc as plsc

VOCAB = 1000000
AUX_VOCAB = 100000
DIM = 64
AUX_DIM = 32
B = 16384
NNEG = 5
EPS = 1e-05

NC = 2
NS = 16
L = 16
NW = NC * NS
S_PER_W = B // NW
C = 64
NCHUNK = S_PER_W // C
NG = C // L
ND = DIM // L
NAD = AUX_DIM // L


def _lane_sum(p, rots):
    for r in rots:
        p = p + jnp.take(p, r)
    return p


def _sc_body(pos_u, pos_v, neg_v, apos_u, apos_v, aneg_v,
             uv_tab, aux_tab,
             pos_out, aux_out, neg_out, auxneg_out,
             iu, iv, ineg, iau, iav, ianeg,
             ur, vr, nr, aur, avr, anr,
             dots_pos, dots_aux, dots_neg, dots_auxneg,
             sem):
    wid = lax.axis_index("s") * NC + lax.axis_index("c")
    base = wid * S_PER_W
    lane = lax.iota(jnp.int32, L)
    rots = [(lane + sh) % L for sh in (8, 4, 2, 1)]
    zero = jnp.zeros((L,), jnp.float32)

    def chunk(ck, _):
        off = base + ck * C

        pltpu.sync_copy(pos_u.at[pl.ds(off, C)], iu)
        pltpu.sync_copy(pos_v.at[pl.ds(off, C)], iv)
        pltpu.sync_copy(neg_v.at[pl.ds(off * NNEG, C * NNEG)], ineg)
        pltpu.sync_copy(apos_u.at[pl.ds(off, C)], iau)
        pltpu.sync_copy(apos_v.at[pl.ds(off, C)], iav)
        pltpu.sync_copy(aneg_v.at[pl.ds(off * NNEG, C * NNEG)], ianeg)

        cps = [pltpu.async_copy(uv_tab.at[iu], ur, sem),
               pltpu.async_copy(uv_tab.at[iv], vr, sem),
               pltpu.async_copy(aux_tab.at[iau], aur, sem),
               pltpu.async_copy(aux_tab.at[iav], avr, sem)]
        for r in range(NNEG):
            cps.append(pltpu.async_copy(
                uv_tab.at[ineg.at[pl.ds(r * C, C)]],
                nr.at[pl.ds(r * C, C)], sem))
            cps.append(pltpu.async_copy(
                aux_tab.at[ianeg.at[pl.ds(r * C, C)]],
                anr.at[pl.ds(r * C, C)], sem))
        for cp in cps:
            cp.wait()

        def group(g, _):
            def samp(i, accs):
                apos, aaux, aneg, aaneg = accs
                s = g * L + i
                sel = lane == i
                urow = ur.at[s]
                vrow = vr.at[s]
                us = [urow[pl.ds(k * L, L)] for k in range(ND)]
                p = us[0] * vrow[pl.ds(DIM, L)]
                for k in range(1, ND):
                    p = p + us[k] * vrow[pl.ds(DIM + k * L, L)]
                apos = jnp.where(sel, _lane_sum(p, rots), apos)

                aurow = aur.at[s]
                avrow = avr.at[s]
                aus = [aurow[pl.ds(k * L, L)] for k in range(NAD)]
                a = aus[0] * avrow[pl.ds(AUX_DIM, L)]
                for k in range(1, NAD):
                    a = a + aus[k] * avrow[pl.ds(AUX_DIM + k * L, L)]
                aaux = jnp.where(sel, _lane_sum(a, rots), aaux)

                aneg2, aaneg2 = [], []
                for n in range(NNEG):
                    nrow = nr.at[n * C + s]
                    q = us[0] * nrow[pl.ds(DIM, L)]
                    for k in range(1, ND):
                        q = q + us[k] * nrow[pl.ds(DIM + k * L, L)]
                    aneg2.append(jnp.where(sel, _lane_sum(q, rots), aneg[n]))

                    anrow = anr.at[n * C + s]
                    aq = aus[0] * anrow[pl.ds(AUX_DIM, L)]
                    for k in range(1, NAD):
                        aq = aq + aus[k] * anrow[pl.ds(AUX_DIM + k * L, L)]
                    aaneg2.append(
                        jnp.where(sel, _lane_sum(aq, rots), aaneg[n]))
                return apos, aaux, tuple(aneg2), tuple(aaneg2)

            init = (zero, zero, (zero,) * NNEG, (zero,) * NNEG)
            apos, aaux, aneg, aaneg = lax.fori_loop(0, L, samp, init)

            dots_pos[pl.ds(g * L, L)] = apos
            dots_aux[pl.ds(g * L, L)] = aaux
            for n in range(NNEG):
                dots_neg[pl.ds(n * C + g * L, L)] = aneg[n]
                dots_auxneg[pl.ds(n * C + g * L, L)] = aaneg[n]
            return 0

        lax.fori_loop(0, NG, group, 0)

        pltpu.sync_copy(dots_pos, pos_out.at[pl.ds(off, C)])
        pltpu.sync_copy(dots_aux, aux_out.at[pl.ds(off, C)])
        for n in range(NNEG):
            pltpu.sync_copy(dots_neg.at[pl.ds(n * C, C)],
                            neg_out.at[pl.ds(n * B + off, C)])
            pltpu.sync_copy(dots_auxneg.at[pl.ds(n * C, C)],
                            auxneg_out.at[pl.ds(n * B + off, C)])
        return 0

    lax.fori_loop(0, NCHUNK, chunk, 0)


@functools.cache
def _make_sc_dots():
  return functools.partial(
    pl.kernel,
    out_type=[
        jax.ShapeDtypeStruct((B,), jnp.float32),
        jax.ShapeDtypeStruct((B,), jnp.float32),
        jax.ShapeDtypeStruct((NNEG * B,), jnp.float32),
        jax.ShapeDtypeStruct((NNEG * B,), jnp.float32),
    ],
    mesh=plsc.VectorSubcoreMesh(core_axis_name="c", subcore_axis_name="s",
                                num_cores=NC, num_subcores=NS),
    scratch_types=[
        pltpu.VMEM((C,), jnp.int32),
        pltpu.VMEM((C,), jnp.int32),
        pltpu.VMEM((C * NNEG,), jnp.int32),
        pltpu.VMEM((C,), jnp.int32),
        pltpu.VMEM((C,), jnp.int32),
        pltpu.VMEM((C * NNEG,), jnp.int32),
        pltpu.VMEM((C, 2 * DIM), jnp.float32),
        pltpu.VMEM((C, 2 * DIM), jnp.float32),
        pltpu.VMEM((C * NNEG, 2 * DIM), jnp.float32),
        pltpu.VMEM((C, 4 * AUX_DIM), jnp.float32),
        pltpu.VMEM((C, 4 * AUX_DIM), jnp.float32),
        pltpu.VMEM((C * NNEG, 4 * AUX_DIM), jnp.float32),
        pltpu.VMEM((C,), jnp.float32),
        pltpu.VMEM((C,), jnp.float32),
        pltpu.VMEM((C * NNEG,), jnp.float32),
        pltpu.VMEM((C * NNEG,), jnp.float32),
        pltpu.SemaphoreType.DMA,
    ],
  )(_sc_body)


def _pack_body(ut_ref, vt_ref, o_ref):
    d = ut_ref.shape[0]
    eye = (lax.broadcasted_iota(jnp.int32, (d, d), 0)
           == lax.broadcasted_iota(jnp.int32, (d, d), 1)).astype(jnp.float32)
    dn = (((0,), (0,)), ((), ()))
    xu = lax.dot_general(ut_ref[...], eye, dn,
                         preferred_element_type=jnp.float32)
    xv = lax.dot_general(vt_ref[...], eye, dn,
                         preferred_element_type=jnp.float32)
    reps = 128 // (2 * d)
    o_ref[...] = jnp.concatenate([xu, xv] * reps, axis=1)


def _pack_tables(ut, vt, w):
    d, v = ut.shape
    grid = (v + w - 1) // w
    return pl.pallas_call(
        _pack_body,
        grid=(grid,),
        in_specs=[pl.BlockSpec((d, w), lambda j: (0, j)),
                  pl.BlockSpec((d, w), lambda j: (0, j))],
        out_specs=pl.BlockSpec((w, 128), lambda j: (j, 0)),
        out_shape=jax.ShapeDtypeStruct((v, 128), jnp.float32),
    )(ut, vt)


def _tc_body(pos_ref, aux_ref, neg_ref, aneg_ref, o1_ref, o2_ref):
    pos = pos_ref[...]
    f1 = -jnp.log(jnp.clip(pos, EPS, 1.0 - EPS))
    neg = neg_ref[...]
    g1 = jnp.log(1.0 - jnp.clip(neg, EPS, 1.0 - EPS))
    o1_ref[...] = jnp.reshape((jnp.sum(f1) - jnp.sum(g1)) / B, (1, 1))

    aux = aux_ref[...]
    f2 = -jnp.log(jnp.clip(aux, EPS, 1.0 - EPS))
    an = aneg_ref[...]
    g2 = jnp.log(1.0 - jnp.clip(an, EPS, 1.0 - EPS))
    o2_ref[...] = jnp.reshape((jnp.sum(f2) - jnp.sum(g2)) / B, (1, 1))


def _tc_loss(pos_d, aux_d, neg_d, aneg_d):
    return pl.pallas_call(
        _tc_body,
        out_shape=[jax.ShapeDtypeStruct((1, 1), jnp.float32),
                   jax.ShapeDtypeStruct((1, 1), jnp.float32)],
    )(pos_d, aux_d, neg_d, aneg_d)


def kernel(pos_u, pos_v, neg_v, aux_pos_u, aux_pos_v, aux_neg_v,
           u_emb, v_emb, aux_u_emb, aux_v_emb):
    pos_u = pos_u.astype(jnp.int32)
    pos_v = pos_v.astype(jnp.int32)
    neg_flat = neg_v.reshape(-1).astype(jnp.int32)
    aux_pos_u = aux_pos_u.astype(jnp.int32)
    aux_pos_v = aux_pos_v.astype(jnp.int32)
    aneg_flat = aux_neg_v.reshape(-1).astype(jnp.int32)

    uv_tab = _pack_tables(u_emb.T, v_emb.T, 4096)
    aux_tab = _pack_tables(aux_u_emb.T, aux_v_emb.T, 4096)

    pos_d, aux_d, neg_d, aneg_d = _make_sc_dots()(
        pos_u, pos_v, neg_flat, aux_pos_u, aux_pos_v, aneg_flat,
        uv_tab, aux_tab)

    o1, o2 = _tc_loss(pos_d.reshape(B // 128, 128),
                      aux_d.reshape(B // 128, 128),
                      neg_d.reshape(NNEG * B // 128, 128),
                      aneg_d.reshape(NNEG * B // 128, 128))
    return (o1[0, 0], o2[0, 0])


# speedup vs baseline: 3.3119x; 1.1227x over previous
import functools

import jax
import jax.numpy as jnp
from jax import lax
from jax.experimental import pallas as pl
from jax.experimental.pallas import tpu as pltpu
from jax.experimental.pallas import tpu_sc as plsc

VOCAB = 1000000
AUX_VOCAB = 100000
DIM = 64
AUX_DIM = 32
B = 16384
NNEG = 5
EPS = 1e-05

NC = 2
NS = 16
L = 16
NW = NC * NS
S_PER_W = B // NW
C = 64
NCHUNK = S_PER_W // C
NG = C // L
ND = DIM // L
NAD = AUX_DIM // L


def _lane_sum(p, rots):
    for r in rots:
        p = p + jnp.take(p, r)
    return p


def _sc_body(pos_u, pos_v, neg_v, apos_u, apos_v, aneg_v,
             uv_tab, aux_tab,
             pos_out, aux_out, neg_out, auxneg_out,
             iu, iv, ineg, iau, iav, ianeg,
             ur, vr, nr, aur, avr, anr,
             dots_pos, dots_aux, dots_neg, dots_auxneg,
             sem):
    wid = lax.axis_index("s") * NC + lax.axis_index("c")
    base = wid * S_PER_W
    lane = lax.iota(jnp.int32, L)
    rots = [(lane + sh) % L for sh in (8, 4, 2, 1)]
    zero = jnp.zeros((L,), jnp.float32)

    def chunk(ck, _):
        off = base + ck * C

        pltpu.sync_copy(pos_u.at[pl.ds(off, C)], iu)
        pltpu.sync_copy(pos_v.at[pl.ds(off, C)], iv)
        pltpu.sync_copy(neg_v.at[pl.ds(off * NNEG, C * NNEG)], ineg)
        pltpu.sync_copy(apos_u.at[pl.ds(off, C)], iau)
        pltpu.sync_copy(apos_v.at[pl.ds(off, C)], iav)
        pltpu.sync_copy(aneg_v.at[pl.ds(off * NNEG, C * NNEG)], ianeg)

        cps = [pltpu.async_copy(uv_tab.at[iu], ur, sem),
               pltpu.async_copy(uv_tab.at[iv], vr, sem),
               pltpu.async_copy(aux_tab.at[iau], aur, sem),
               pltpu.async_copy(aux_tab.at[iav], avr, sem)]
        for r in range(NNEG):
            cps.append(pltpu.async_copy(
                uv_tab.at[ineg.at[pl.ds(r * C, C)]],
                nr.at[pl.ds(r * C, C)], sem))
            cps.append(pltpu.async_copy(
                aux_tab.at[ianeg.at[pl.ds(r * C, C)]],
                anr.at[pl.ds(r * C, C)], sem))
        for cp in cps:
            cp.wait()

        def group(g, _):
            def samp(i, accs):
                apos, aaux, aneg, aaneg = accs
                s = g * L + i
                sel = lane == i
                urow = ur.at[s]
                vrow = vr.at[s]
                us = [urow[pl.ds(k * L, L)] for k in range(ND)]
                p = us[0] * vrow[pl.ds(DIM, L)]
                for k in range(1, ND):
                    p = p + us[k] * vrow[pl.ds(DIM + k * L, L)]
                apos = jnp.where(sel, _lane_sum(p, rots), apos)

                aurow = aur.at[s]
                avrow = avr.at[s]
                aus = [aurow[pl.ds(k * L, L)] for k in range(NAD)]
                a = aus[0] * avrow[pl.ds(AUX_DIM, L)]
                for k in range(1, NAD):
                    a = a + aus[k] * avrow[pl.ds(AUX_DIM + k * L, L)]
                aaux = jnp.where(sel, _lane_sum(a, rots), aaux)

                aneg2, aaneg2 = [], []
                for n in range(NNEG):
                    nrow = nr.at[n * C + s]
                    q = us[0] * nrow[pl.ds(DIM, L)]
                    for k in range(1, ND):
                        q = q + us[k] * nrow[pl.ds(DIM + k * L, L)]
                    aneg2.append(jnp.where(sel, _lane_sum(q, rots), aneg[n]))

                    anrow = anr.at[n * C + s]
                    aq = aus[0] * anrow[pl.ds(AUX_DIM, L)]
                    for k in range(1, NAD):
                        aq = aq + aus[k] * anrow[pl.ds(AUX_DIM + k * L, L)]
                    aaneg2.append(
                        jnp.where(sel, _lane_sum(aq, rots), aaneg[n]))
                return apos, aaux, tuple(aneg2), tuple(aaneg2)

            init = (zero, zero, (zero,) * NNEG, (zero,) * NNEG)
            apos, aaux, aneg, aaneg = lax.fori_loop(0, L, samp, init)

            dots_pos[pl.ds(g * L, L)] = apos
            dots_aux[pl.ds(g * L, L)] = aaux
            for n in range(NNEG):
                dots_neg[pl.ds(n * C + g * L, L)] = aneg[n]
                dots_auxneg[pl.ds(n * C + g * L, L)] = aaneg[n]
            return 0

        lax.fori_loop(0, NG, group, 0)

        pltpu.sync_copy(dots_pos, pos_out.at[pl.ds(off, C)])
        pltpu.sync_copy(dots_aux, aux_out.at[pl.ds(off, C)])
        for n in range(NNEG):
            pltpu.sync_copy(dots_neg.at[pl.ds(n * C, C)],
                            neg_out.at[pl.ds(n * B + off, C)])
            pltpu.sync_copy(dots_auxneg.at[pl.ds(n * C, C)],
                            auxneg_out.at[pl.ds(n * B + off, C)])
        return 0

    lax.fori_loop(0, NCHUNK, chunk, 0)


@functools.cache
def _make_sc_dots():
  return functools.partial(
    pl.kernel,
    out_type=[
        jax.ShapeDtypeStruct((B,), jnp.float32),
        jax.ShapeDtypeStruct((B,), jnp.float32),
        jax.ShapeDtypeStruct((NNEG * B,), jnp.float32),
        jax.ShapeDtypeStruct((NNEG * B,), jnp.float32),
    ],
    mesh=plsc.VectorSubcoreMesh(core_axis_name="c", subcore_axis_name="s",
                                num_cores=NC, num_subcores=NS),
    scratch_types=[
        pltpu.VMEM((C,), jnp.int32),
        pltpu.VMEM((C,), jnp.int32),
        pltpu.VMEM((C * NNEG,), jnp.int32),
        pltpu.VMEM((C,), jnp.int32),
        pltpu.VMEM((C,), jnp.int32),
        pltpu.VMEM((C * NNEG,), jnp.int32),
        pltpu.VMEM((C, 2 * DIM), jnp.float32),
        pltpu.VMEM((C, 2 * DIM), jnp.float32),
        pltpu.VMEM((C * NNEG, 2 * DIM), jnp.float32),
        pltpu.VMEM((C, 4 * AUX_DIM), jnp.float32),
        pltpu.VMEM((C, 4 * AUX_DIM), jnp.float32),
        pltpu.VMEM((C * NNEG, 4 * AUX_DIM), jnp.float32),
        pltpu.VMEM((C,), jnp.float32),
        pltpu.VMEM((C,), jnp.float32),
        pltpu.VMEM((C * NNEG,), jnp.float32),
        pltpu.VMEM((C * NNEG,), jnp.float32),
        pltpu.SemaphoreType.DMA,
    ],
  )(_sc_body)


def _pack_body(ut_ref, vt_ref, o_ref):
    d = ut_ref.shape[0]
    iod = lax.broadcasted_iota(jnp.int32, (d, 128), 0)
    ioc = lax.broadcasted_iota(jnp.int32, (d, 128), 1)
    e_lo = (ioc == iod).astype(jnp.float32)
    e_hi = (ioc == iod + d).astype(jnp.float32)
    dn = (((0,), (0,)), ((), ()))
    o_ref[...] = (
        lax.dot_general(ut_ref[...], e_lo, dn,
                        preferred_element_type=jnp.float32)
        + lax.dot_general(vt_ref[...], e_hi, dn,
                          preferred_element_type=jnp.float32))


def _pack_tables(ut, vt, w):
    d, v = ut.shape
    grid = (v + w - 1) // w
    return pl.pallas_call(
        _pack_body,
        grid=(grid,),
        in_specs=[pl.BlockSpec((d, w), lambda j: (0, j)),
                  pl.BlockSpec((d, w), lambda j: (0, j))],
        out_specs=pl.BlockSpec((w, 128), lambda j: (j, 0)),
        out_shape=jax.ShapeDtypeStruct((v, 128), jnp.float32),
        compiler_params=pltpu.CompilerParams(
            fuse_transposed_lhs_in_matmul=True),
    )(ut, vt)


def _tc_body(pos_ref, aux_ref, neg_ref, aneg_ref, o1_ref, o2_ref):
    pos = pos_ref[...]
    f1 = -jnp.log(jnp.clip(pos, EPS, 1.0 - EPS))
    neg = neg_ref[...]
    g1 = jnp.log(1.0 - jnp.clip(neg, EPS, 1.0 - EPS))
    o1_ref[...] = jnp.reshape((jnp.sum(f1) - jnp.sum(g1)) / B, (1, 1))

    aux = aux_ref[...]
    f2 = -jnp.log(jnp.clip(aux, EPS, 1.0 - EPS))
    an = aneg_ref[...]
    g2 = jnp.log(1.0 - jnp.clip(an, EPS, 1.0 - EPS))
    o2_ref[...] = jnp.reshape((jnp.sum(f2) - jnp.sum(g2)) / B, (1, 1))


def _tc_loss(pos_d, aux_d, neg_d, aneg_d):
    return pl.pallas_call(
        _tc_body,
        out_shape=[jax.ShapeDtypeStruct((1, 1), jnp.float32),
                   jax.ShapeDtypeStruct((1, 1), jnp.float32)],
    )(pos_d, aux_d, neg_d, aneg_d)


def kernel(pos_u, pos_v, neg_v, aux_pos_u, aux_pos_v, aux_neg_v,
           u_emb, v_emb, aux_u_emb, aux_v_emb):
    pos_u = pos_u.astype(jnp.int32)
    pos_v = pos_v.astype(jnp.int32)
    neg_flat = neg_v.reshape(-1).astype(jnp.int32)
    aux_pos_u = aux_pos_u.astype(jnp.int32)
    aux_pos_v = aux_pos_v.astype(jnp.int32)
    aneg_flat = aux_neg_v.reshape(-1).astype(jnp.int32)

    uv_tab = _pack_tables(u_emb.T, v_emb.T, 4096)
    aux_tab = _pack_tables(aux_u_emb.T, aux_v_emb.T, 4096)

    pos_d, aux_d, neg_d, aneg_d = _make_sc_dots()(
        pos_u, pos_v, neg_flat, aux_pos_u, aux_pos_v, aneg_flat,
        uv_tab, aux_tab)

    o1, o2 = _tc_loss(pos_d.reshape(B // 128, 128),
                      aux_d.reshape(B // 128, 128),
                      neg_d.reshape(NNEG * B // 128, 128),
                      aneg_d.reshape(NNEG * B // 128, 128))
    return (o1[0, 0], o2[0, 0])


# speedup vs baseline: 3.7572x; 1.1345x over previous
import functools

import jax
import jax.numpy as jnp
from jax import lax
from jax.experimental import pallas as pl
from jax.experimental.pallas import tpu as pltpu
from jax.experimental.pallas import tpu_sc as plsc

VOCAB = 1000000
AUX_VOCAB = 100000
DIM = 64
AUX_DIM = 32
B = 16384
NNEG = 5
EPS = 1e-05

NC = 2
NS = 16
L = 16
NW = NC * NS
S_PER_W = B // NW
C = 32
NCHUNK = S_PER_W // C
NG = C // L
ND = DIM // L
NAD = AUX_DIM // L


def _lane_sum(p, rots):
    for r in rots:
        p = p + jnp.take(p, r)
    return p


def _sc_body(pos_u, pos_v, neg_v, apos_u, apos_v, aneg_v,
             uv_tab, aux_tab,
             pos_out, aux_out, neg_out, auxneg_out,
             iu, iv, ineg, iau, iav, ianeg,
             ur0, vr0, nr0, aur0, avr0, anr0,
             ur1, vr1, nr1, aur1, avr1, anr1,
             dots_pos, dots_aux, dots_neg, dots_auxneg,
             sem):
    wid = lax.axis_index("s") * NC + lax.axis_index("c")
    base = wid * S_PER_W
    lane = lax.iota(jnp.int32, L)
    rots = [(lane + sh) % L for sh in (8, 4, 2, 1)]
    zero = jnp.zeros((L,), jnp.float32)
    bufs = [(ur0, vr0, nr0, aur0, avr0, anr0),
            (ur1, vr1, nr1, aur1, avr1, anr1)]

    def stage_and_fire(ck, bi):
        off = base + ck * C
        ur, vr, nr, aur, avr, anr = bufs[bi]
        pltpu.sync_copy(pos_u.at[pl.ds(off, C)], iu)
        pltpu.sync_copy(pos_v.at[pl.ds(off, C)], iv)
        pltpu.sync_copy(neg_v.at[pl.ds(off * NNEG, C * NNEG)], ineg)
        pltpu.sync_copy(apos_u.at[pl.ds(off, C)], iau)
        pltpu.sync_copy(apos_v.at[pl.ds(off, C)], iav)
        pltpu.sync_copy(aneg_v.at[pl.ds(off * NNEG, C * NNEG)], ianeg)
        cps = [pltpu.async_copy(uv_tab.at[iu], ur, sem),
               pltpu.async_copy(uv_tab.at[iv], vr, sem),
               pltpu.async_copy(aux_tab.at[iau], aur, sem),
               pltpu.async_copy(aux_tab.at[iav], avr, sem)]
        for r in range(NNEG):
            cps.append(pltpu.async_copy(
                uv_tab.at[ineg.at[pl.ds(r * C, C)]],
                nr.at[pl.ds(r * C, C)], sem))
            cps.append(pltpu.async_copy(
                aux_tab.at[ianeg.at[pl.ds(r * C, C)]],
                anr.at[pl.ds(r * C, C)], sem))
        return cps

    def compute(ck, bi):
        off = base + ck * C
        ur, vr, nr, aur, avr, anr = bufs[bi]

        def group(g, _):
            def samp(i, accs):
                apos, aaux, aneg, aaneg = accs
                s = g * L + i
                sel = lane == i
                urow = ur.at[s]
                vrow = vr.at[s]
                us = [urow[pl.ds(k * L, L)] for k in range(ND)]
                p = us[0] * vrow[pl.ds(DIM, L)]
                for k in range(1, ND):
                    p = p + us[k] * vrow[pl.ds(DIM + k * L, L)]
                apos = jnp.where(sel, _lane_sum(p, rots), apos)

                aurow = aur.at[s]
                avrow = avr.at[s]
                aus = [aurow[pl.ds(k * L, L)] for k in range(NAD)]
                a = aus[0] * avrow[pl.ds(AUX_DIM, L)]
                for k in range(1, NAD):
                    a = a + aus[k] * avrow[pl.ds(AUX_DIM + k * L, L)]
                aaux = jnp.where(sel, _lane_sum(a, rots), aaux)

                aneg2, aaneg2 = [], []
                for n in range(NNEG):
                    nrow = nr.at[n * C + s]
                    q = us[0] * nrow[pl.ds(DIM, L)]
                    for k in range(1, ND):
                        q = q + us[k] * nrow[pl.ds(DIM + k * L, L)]
                    aneg2.append(jnp.where(sel, _lane_sum(q, rots), aneg[n]))

                    anrow = anr.at[n * C + s]
                    aq = aus[0] * anrow[pl.ds(AUX_DIM, L)]
                    for k in range(1, NAD):
                        aq = aq + aus[k] * anrow[pl.ds(AUX_DIM + k * L, L)]
                    aaneg2.append(
                        jnp.where(sel, _lane_sum(aq, rots), aaneg[n]))
                return apos, aaux, tuple(aneg2), tuple(aaneg2)

            init = (zero, zero, (zero,) * NNEG, (zero,) * NNEG)
            apos, aaux, aneg, aaneg = lax.fori_loop(0, L, samp, init)

            dots_pos[pl.ds(g * L, L)] = apos
            dots_aux[pl.ds(g * L, L)] = aaux
            for n in range(NNEG):
                dots_neg[pl.ds(n * C + g * L, L)] = aneg[n]
                dots_auxneg[pl.ds(n * C + g * L, L)] = aaneg[n]
            return 0

        lax.fori_loop(0, NG, group, 0)

        pltpu.sync_copy(dots_pos, pos_out.at[pl.ds(off, C)])
        pltpu.sync_copy(dots_aux, aux_out.at[pl.ds(off, C)])
        for n in range(NNEG):
            pltpu.sync_copy(dots_neg.at[pl.ds(n * C, C)],
                            neg_out.at[pl.ds(n * B + off, C)])
            pltpu.sync_copy(dots_auxneg.at[pl.ds(n * C, C)],
                            auxneg_out.at[pl.ds(n * B + off, C)])

    cps = stage_and_fire(0, 0)
    for ck in range(NCHUNK):
        for cp in cps:
            cp.wait()
        cps = stage_and_fire(ck + 1, (ck + 1) % 2) if ck + 1 < NCHUNK else []
        compute(ck, ck % 2)


@functools.cache
def _make_sc_dots():
  return functools.partial(
    pl.kernel,
    out_type=[
        jax.ShapeDtypeStruct((B,), jnp.float32),
        jax.ShapeDtypeStruct((B,), jnp.float32),
        jax.ShapeDtypeStruct((NNEG * B,), jnp.float32),
        jax.ShapeDtypeStruct((NNEG * B,), jnp.float32),
    ],
    mesh=plsc.VectorSubcoreMesh(core_axis_name="c", subcore_axis_name="s",
                                num_cores=NC, num_subcores=NS),
    scratch_types=[
        pltpu.VMEM((C,), jnp.int32),
        pltpu.VMEM((C,), jnp.int32),
        pltpu.VMEM((C * NNEG,), jnp.int32),
        pltpu.VMEM((C,), jnp.int32),
        pltpu.VMEM((C,), jnp.int32),
        pltpu.VMEM((C * NNEG,), jnp.int32),
        pltpu.VMEM((C, 2 * DIM), jnp.float32),
        pltpu.VMEM((C, 2 * DIM), jnp.float32),
        pltpu.VMEM((C * NNEG, 2 * DIM), jnp.float32),
        pltpu.VMEM((C, 4 * AUX_DIM), jnp.float32),
        pltpu.VMEM((C, 4 * AUX_DIM), jnp.float32),
        pltpu.VMEM((C * NNEG, 4 * AUX_DIM), jnp.float32),
        pltpu.VMEM((C, 2 * DIM), jnp.float32),
        pltpu.VMEM((C, 2 * DIM), jnp.float32),
        pltpu.VMEM((C * NNEG, 2 * DIM), jnp.float32),
        pltpu.VMEM((C, 4 * AUX_DIM), jnp.float32),
        pltpu.VMEM((C, 4 * AUX_DIM), jnp.float32),
        pltpu.VMEM((C * NNEG, 4 * AUX_DIM), jnp.float32),
        pltpu.VMEM((C,), jnp.float32),
        pltpu.VMEM((C,), jnp.float32),
        pltpu.VMEM((C * NNEG,), jnp.float32),
        pltpu.VMEM((C * NNEG,), jnp.float32),
        pltpu.SemaphoreType.DMA,
    ],
  )(_sc_body)


def _pack_body(ut_ref, vt_ref, o_ref):
    d = ut_ref.shape[0]
    iod = lax.broadcasted_iota(jnp.int32, (d, 128), 0)
    ioc = lax.broadcasted_iota(jnp.int32, (d, 128), 1)
    e_lo = (ioc == iod).astype(jnp.float32)
    e_hi = (ioc == iod + d).astype(jnp.float32)
    dn = (((0,), (0,)), ((), ()))
    o_ref[...] = (
        lax.dot_general(ut_ref[...], e_lo, dn,
                        preferred_element_type=jnp.float32)
        + lax.dot_general(vt_ref[...], e_hi, dn,
                          preferred_element_type=jnp.float32))


def _pack_tables(ut, vt, w):
    d, v = ut.shape
    grid = (v + w - 1) // w
    return pl.pallas_call(
        _pack_body,
        grid=(grid,),
        in_specs=[pl.BlockSpec((d, w), lambda j: (0, j)),
                  pl.BlockSpec((d, w), lambda j: (0, j))],
        out_specs=pl.BlockSpec((w, 128), lambda j: (j, 0)),
        out_shape=jax.ShapeDtypeStruct((v, 128), jnp.float32),
        compiler_params=pltpu.CompilerParams(
            fuse_transposed_lhs_in_matmul=True),
    )(ut, vt)


def _tc_body(pos_ref, aux_ref, neg_ref, aneg_ref, o1_ref, o2_ref):
    pos = pos_ref[...]
    f1 = -jnp.log(jnp.clip(pos, EPS, 1.0 - EPS))
    neg = neg_ref[...]
    g1 = jnp.log(1.0 - jnp.clip(neg, EPS, 1.0 - EPS))
    o1_ref[...] = jnp.reshape((jnp.sum(f1) - jnp.sum(g1)) / B, (1, 1))

    aux = aux_ref[...]
    f2 = -jnp.log(jnp.clip(aux, EPS, 1.0 - EPS))
    an = aneg_ref[...]
    g2 = jnp.log(1.0 - jnp.clip(an, EPS, 1.0 - EPS))
    o2_ref[...] = jnp.reshape((jnp.sum(f2) - jnp.sum(g2)) / B, (1, 1))


def _tc_loss(pos_d, aux_d, neg_d, aneg_d):
    return pl.pallas_call(
        _tc_body,
        out_shape=[jax.ShapeDtypeStruct((1, 1), jnp.float32),
                   jax.ShapeDtypeStruct((1, 1), jnp.float32)],
    )(pos_d, aux_d, neg_d, aneg_d)


def kernel(pos_u, pos_v, neg_v, aux_pos_u, aux_pos_v, aux_neg_v,
           u_emb, v_emb, aux_u_emb, aux_v_emb):
    pos_u = pos_u.astype(jnp.int32)
    pos_v = pos_v.astype(jnp.int32)
    neg_flat = neg_v.reshape(-1).astype(jnp.int32)
    aux_pos_u = aux_pos_u.astype(jnp.int32)
    aux_pos_v = aux_pos_v.astype(jnp.int32)
    aneg_flat = aux_neg_v.reshape(-1).astype(jnp.int32)

    uv_tab = _pack_tables(u_emb.T, v_emb.T, 8192)
    aux_tab = _pack_tables(aux_u_emb.T, aux_v_emb.T, 8192)

    pos_d, aux_d, neg_d, aneg_d = _make_sc_dots()(
        pos_u, pos_v, neg_flat, aux_pos_u, aux_pos_v, aneg_flat,
        uv_tab, aux_tab)

    o1, o2 = _tc_loss(pos_d.reshape(B // 128, 128),
                      aux_d.reshape(B // 128, 128),
                      neg_d.reshape(NNEG * B // 128, 128),
                      aneg_d.reshape(NNEG * B // 128, 128))
    return (o1[0, 0], o2[0, 0])


# speedup vs baseline: 4.0257x; 1.0715x over previous
import functools

import jax
import jax.numpy as jnp
from jax import lax
from jax.experimental import pallas as pl
from jax.experimental.pallas import tpu as pltpu
from jax.experimental.pallas import tpu_sc as plsc

VOCAB = 1000000
AUX_VOCAB = 100000
DIM = 64
AUX_DIM = 32
B = 16384
NNEG = 5
EPS = 1e-05

NC = 2
NS = 16
L = 16
NW = NC * NS
S_PER_W = B // NW
C = 32
NCHUNK = S_PER_W // C
NG = C // L
ND = DIM // L
NAD = AUX_DIM // L


def _lane_sum(p, rots):
    for r in rots:
        p = p + jnp.take(p, r)
    return p


def _sc_body(pos_u, pos_v, neg_v, apos_u, apos_v, aneg_v,
             uv_tab, aux_tab,
             pos_out, aux_out, neg_out, auxneg_out,
             iu0, iv0, ineg0, iau0, iav0, ianeg0,
             iu1, iv1, ineg1, iau1, iav1, ianeg1,
             ur0, vr0, nr0, aur0, avr0, anr0,
             ur1, vr1, nr1, aur1, avr1, anr1,
             dp0, da0, dn0, dan0, dp1, da1, dn1, dan1,
             semi0, semi1, semg0, semg1, semd0, semd1):
    wid = lax.axis_index("s") * NC + lax.axis_index("c")
    base = wid * S_PER_W
    lane = lax.iota(jnp.int32, L)
    rots = [(lane + sh) % L for sh in (8, 4, 2, 1)]
    zero = jnp.zeros((L,), jnp.float32)
    bufs = [(ur0, vr0, nr0, aur0, avr0, anr0),
            (ur1, vr1, nr1, aur1, avr1, anr1)]
    ibufs = [(iu0, iv0, ineg0, iau0, iav0, ianeg0),
             (iu1, iv1, ineg1, iau1, iav1, ianeg1)]
    dbufs = [(dp0, da0, dn0, dan0), (dp1, da1, dn1, dan1)]
    semi = [semi0, semi1]
    semg = [semg0, semg1]
    semd = [semd0, semd1]

    def stage(ck):
        off = base + ck * C
        iu, iv, ineg, iau, iav, ianeg = ibufs[ck % 2]
        sem = semi[ck % 2]
        return [pltpu.async_copy(pos_u.at[pl.ds(off, C)], iu, sem),
                pltpu.async_copy(pos_v.at[pl.ds(off, C)], iv, sem),
                pltpu.async_copy(neg_v.at[pl.ds(off * NNEG, C * NNEG)],
                                 ineg, sem),
                pltpu.async_copy(apos_u.at[pl.ds(off, C)], iau, sem),
                pltpu.async_copy(apos_v.at[pl.ds(off, C)], iav, sem),
                pltpu.async_copy(aneg_v.at[pl.ds(off * NNEG, C * NNEG)],
                                 ianeg, sem)]

    def fire(ck):
        ur, vr, nr, aur, avr, anr = bufs[ck % 2]
        iu, iv, ineg, iau, iav, ianeg = ibufs[ck % 2]
        sem = semg[ck % 2]
        cps = [pltpu.async_copy(uv_tab.at[iu], ur, sem),
               pltpu.async_copy(uv_tab.at[iv], vr, sem),
               pltpu.async_copy(aux_tab.at[iau], aur, sem),
               pltpu.async_copy(aux_tab.at[iav], avr, sem)]
        for r in range(NNEG):
            cps.append(pltpu.async_copy(
                uv_tab.at[ineg.at[pl.ds(r * C, C)]],
                nr.at[pl.ds(r * C, C)], sem))
            cps.append(pltpu.async_copy(
                aux_tab.at[ianeg.at[pl.ds(r * C, C)]],
                anr.at[pl.ds(r * C, C)], sem))
        return cps

    def compute(ck, bi):
        off = base + ck * C
        ur, vr, nr, aur, avr, anr = bufs[bi]
        dots_pos, dots_aux, dots_neg, dots_auxneg = dbufs[bi]

        def group(g, _):
            def samp(i, accs):
                apos, aaux, aneg, aaneg = accs
                s = g * L + i
                sel = lane == i
                urow = ur.at[s]
                vrow = vr.at[s]
                us = [urow[pl.ds(k * L, L)] for k in range(ND)]
                p = us[0] * vrow[pl.ds(DIM, L)]
                for k in range(1, ND):
                    p = p + us[k] * vrow[pl.ds(DIM + k * L, L)]
                apos = jnp.where(sel, _lane_sum(p, rots), apos)

                aurow = aur.at[s]
                avrow = avr.at[s]
                aus = [aurow[pl.ds(k * L, L)] for k in range(NAD)]
                a = aus[0] * avrow[pl.ds(AUX_DIM, L)]
                for k in range(1, NAD):
                    a = a + aus[k] * avrow[pl.ds(AUX_DIM + k * L, L)]
                aaux = jnp.where(sel, _lane_sum(a, rots), aaux)

                aneg2, aaneg2 = [], []
                for n in range(NNEG):
                    nrow = nr.at[n * C + s]
                    q = us[0] * nrow[pl.ds(DIM, L)]
                    for k in range(1, ND):
                        q = q + us[k] * nrow[pl.ds(DIM + k * L, L)]
                    aneg2.append(jnp.where(sel, _lane_sum(q, rots), aneg[n]))

                    anrow = anr.at[n * C + s]
                    aq = aus[0] * anrow[pl.ds(AUX_DIM, L)]
                    for k in range(1, NAD):
                        aq = aq + aus[k] * anrow[pl.ds(AUX_DIM + k * L, L)]
                    aaneg2.append(
                        jnp.where(sel, _lane_sum(aq, rots), aaneg[n]))
                return apos, aaux, tuple(aneg2), tuple(aaneg2)

            init = (zero, zero, (zero,) * NNEG, (zero,) * NNEG)
            apos, aaux, aneg, aaneg = lax.fori_loop(0, L, samp, init)

            dots_pos[pl.ds(g * L, L)] = apos
            dots_aux[pl.ds(g * L, L)] = aaux
            for n in range(NNEG):
                dots_neg[pl.ds(n * C + g * L, L)] = aneg[n]
                dots_auxneg[pl.ds(n * C + g * L, L)] = aaneg[n]
            return 0

        lax.fori_loop(0, NG, group, 0)

        sem = semd[bi]
        cps = [pltpu.async_copy(dots_pos, pos_out.at[pl.ds(off, C)], sem),
               pltpu.async_copy(dots_aux, aux_out.at[pl.ds(off, C)], sem)]
        for n in range(NNEG):
            cps.append(pltpu.async_copy(
                dots_neg.at[pl.ds(n * C, C)],
                neg_out.at[pl.ds(n * B + off, C)], sem))
            cps.append(pltpu.async_copy(
                dots_auxneg.at[pl.ds(n * C, C)],
                auxneg_out.at[pl.ds(n * B + off, C)], sem))
        return cps

    i_next = stage(0)
    for cp in i_next:
        cp.wait()
    g_cps = fire(0)
    i_next = stage(1) if NCHUNK > 1 else []
    d_prev = [[], []]
    for ck in range(NCHUNK):
        for cp in g_cps:
            cp.wait()
        if ck + 1 < NCHUNK:
            for cp in i_next:
                cp.wait()
            g_cps = fire(ck + 1)
            if ck + 2 < NCHUNK:
                i_next = stage(ck + 2)
        for cp in d_prev[ck % 2]:
            cp.wait()
        d_prev[ck % 2] = compute(ck, ck % 2)
    for par in (0, 1):
        for cp in d_prev[par]:
            cp.wait()


@functools.cache
def _make_sc_dots():
  return functools.partial(
    pl.kernel,
    out_type=[
        jax.ShapeDtypeStruct((B,), jnp.float32),
        jax.ShapeDtypeStruct((B,), jnp.float32),
        jax.ShapeDtypeStruct((NNEG * B,), jnp.float32),
        jax.ShapeDtypeStruct((NNEG * B,), jnp.float32),
    ],
    mesh=plsc.VectorSubcoreMesh(core_axis_name="c", subcore_axis_name="s",
                                num_cores=NC, num_subcores=NS),
    scratch_types=(
        [pltpu.VMEM((C,), jnp.int32),
         pltpu.VMEM((C,), jnp.int32),
         pltpu.VMEM((C * NNEG,), jnp.int32),
         pltpu.VMEM((C,), jnp.int32),
         pltpu.VMEM((C,), jnp.int32),
         pltpu.VMEM((C * NNEG,), jnp.int32)] * 2
        + [pltpu.VMEM((C, 2 * DIM), jnp.float32),
           pltpu.VMEM((C, 2 * DIM), jnp.float32),
           pltpu.VMEM((C * NNEG, 2 * DIM), jnp.float32),
           pltpu.VMEM((C, 4 * AUX_DIM), jnp.float32),
           pltpu.VMEM((C, 4 * AUX_DIM), jnp.float32),
           pltpu.VMEM((C * NNEG, 4 * AUX_DIM), jnp.float32)] * 2
        + [pltpu.VMEM((C,), jnp.float32),
           pltpu.VMEM((C,), jnp.float32),
           pltpu.VMEM((C * NNEG,), jnp.float32),
           pltpu.VMEM((C * NNEG,), jnp.float32)] * 2
        + [pltpu.SemaphoreType.DMA] * 6
    ),
  )(_sc_body)


def _pack_body(ut_ref, vt_ref, o_ref):
    d = ut_ref.shape[0]
    iod = lax.broadcasted_iota(jnp.int32, (d, 128), 0)
    ioc = lax.broadcasted_iota(jnp.int32, (d, 128), 1)
    e_lo = (ioc == iod).astype(jnp.float32)
    e_hi = (ioc == iod + d).astype(jnp.float32)
    dn = (((0,), (0,)), ((), ()))
    o_ref[...] = (
        lax.dot_general(ut_ref[...], e_lo, dn,
                        preferred_element_type=jnp.float32)
        + lax.dot_general(vt_ref[...], e_hi, dn,
                          preferred_element_type=jnp.float32))


def _pack_tables(ut, vt, w):
    d, v = ut.shape
    grid = (v + w - 1) // w
    return pl.pallas_call(
        _pack_body,
        grid=(grid,),
        in_specs=[pl.BlockSpec((d, w), lambda j: (0, j)),
                  pl.BlockSpec((d, w), lambda j: (0, j))],
        out_specs=pl.BlockSpec((w, 128), lambda j: (j, 0)),
        out_shape=jax.ShapeDtypeStruct((v, 128), jnp.float32),
        compiler_params=pltpu.CompilerParams(
            fuse_transposed_lhs_in_matmul=True),
    )(ut, vt)


def _tc_body(pos_ref, aux_ref, neg_ref, aneg_ref, o1_ref, o2_ref):
    pos = pos_ref[...]
    f1 = -jnp.log(jnp.clip(pos, EPS, 1.0 - EPS))
    neg = neg_ref[...]
    g1 = jnp.log(1.0 - jnp.clip(neg, EPS, 1.0 - EPS))
    o1_ref[...] = jnp.reshape((jnp.sum(f1) - jnp.sum(g1)) / B, (1, 1))

    aux = aux_ref[...]
    f2 = -jnp.log(jnp.clip(aux, EPS, 1.0 - EPS))
    an = aneg_ref[...]
    g2 = jnp.log(1.0 - jnp.clip(an, EPS, 1.0 - EPS))
    o2_ref[...] = jnp.reshape((jnp.sum(f2) - jnp.sum(g2)) / B, (1, 1))


def _tc_loss(pos_d, aux_d, neg_d, aneg_d):
    return pl.pallas_call(
        _tc_body,
        out_shape=[jax.ShapeDtypeStruct((1, 1), jnp.float32),
                   jax.ShapeDtypeStruct((1, 1), jnp.float32)],
    )(pos_d, aux_d, neg_d, aneg_d)


def kernel(pos_u, pos_v, neg_v, aux_pos_u, aux_pos_v, aux_neg_v,
           u_emb, v_emb, aux_u_emb, aux_v_emb):
    pos_u = pos_u.astype(jnp.int32)
    pos_v = pos_v.astype(jnp.int32)
    neg_flat = neg_v.reshape(-1).astype(jnp.int32)
    aux_pos_u = aux_pos_u.astype(jnp.int32)
    aux_pos_v = aux_pos_v.astype(jnp.int32)
    aneg_flat = aux_neg_v.reshape(-1).astype(jnp.int32)

    uv_tab = _pack_tables(u_emb.T, v_emb.T, 8192)
    aux_tab = _pack_tables(aux_u_emb.T, aux_v_emb.T, 8192)

    pos_d, aux_d, neg_d, aneg_d = _make_sc_dots()(
        pos_u, pos_v, neg_flat, aux_pos_u, aux_pos_v, aneg_flat,
        uv_tab, aux_tab)

    o1, o2 = _tc_loss(pos_d.reshape(B // 128, 128),
                      aux_d.reshape(B // 128, 128),
                      neg_d.reshape(NNEG * B // 128, 128),
                      aneg_d.reshape(NNEG * B // 128, 128))
    return (o1[0, 0], o2[0, 0])


# speedup vs baseline: 4.3720x; 1.0860x over previous
import functools

import jax
import jax.numpy as jnp
from jax import lax
from jax.experimental import pallas as pl
from jax.experimental.pallas import tpu as pltpu
from jax.experimental.pallas import tpu_sc as plsc

VOCAB = 1000000
AUX_VOCAB = 100000
DIM = 64
AUX_DIM = 32
B = 16384
NNEG = 5
EPS = 1e-05

NC = 2
NS = 16
L = 16
NW = NC * NS
S_PER_W = B // NW
C = 32
NCHUNK = S_PER_W // C
NG = C // L
ND = DIM // L
NAD = AUX_DIM // L


def _lane_sum(p, rots):
    for r in rots:
        p = p + jnp.take(p, r)
    return p


def _sc_body(pos_u, pos_v, neg_v, apos_u, apos_v, aneg_v,
             uv_tab, aux_tab,
             pos_out, aux_out, neg_out, auxneg_out,
             iu0, iv0, ineg0, iau0, iav0, ianeg0,
             iu1, iv1, ineg1, iau1, iav1, ianeg1,
             ur0, vr0, nr0, aur0, avr0, anr0,
             ur1, vr1, nr1, aur1, avr1, anr1,
             dp0, da0, dn0, dan0, dp1, da1, dn1, dan1,
             semi0, semi1, semg0, semg1, semd0, semd1):
    wid = lax.axis_index("s") * NC + lax.axis_index("c")
    base = wid * S_PER_W
    lane = lax.iota(jnp.int32, L)
    rots = [(lane + sh) % L for sh in (8, 4, 2, 1)]
    zero = jnp.zeros((L,), jnp.float32)
    bufs = [(ur0, vr0, nr0, aur0, avr0, anr0),
            (ur1, vr1, nr1, aur1, avr1, anr1)]
    ibufs = [(iu0, iv0, ineg0, iau0, iav0, ianeg0),
             (iu1, iv1, ineg1, iau1, iav1, ianeg1)]
    dbufs = [(dp0, da0, dn0, dan0), (dp1, da1, dn1, dan1)]
    semi = [semi0, semi1]
    semg = [semg0, semg1]
    semd = [semd0, semd1]

    def stage(ck):
        off = base + ck * C
        iu, iv, ineg, iau, iav, ianeg = ibufs[ck % 2]
        sem = semi[ck % 2]
        return [pltpu.async_copy(pos_u.at[pl.ds(off, C)], iu, sem),
                pltpu.async_copy(pos_v.at[pl.ds(off, C)], iv, sem),
                pltpu.async_copy(neg_v.at[pl.ds(off * NNEG, C * NNEG)],
                                 ineg, sem),
                pltpu.async_copy(apos_u.at[pl.ds(off, C)], iau, sem),
                pltpu.async_copy(apos_v.at[pl.ds(off, C)], iav, sem),
                pltpu.async_copy(aneg_v.at[pl.ds(off * NNEG, C * NNEG)],
                                 ianeg, sem)]

    def fire(ck):
        ur, vr, nr, aur, avr, anr = bufs[ck % 2]
        iu, iv, ineg, iau, iav, ianeg = ibufs[ck % 2]
        sem = semg[ck % 2]
        cps = [pltpu.async_copy(uv_tab.at[iu], ur, sem),
               pltpu.async_copy(uv_tab.at[iv], vr, sem),
               pltpu.async_copy(aux_tab.at[iau], aur, sem),
               pltpu.async_copy(aux_tab.at[iav], avr, sem)]
        for r in range(NNEG):
            cps.append(pltpu.async_copy(
                uv_tab.at[ineg.at[pl.ds(r * C, C)]],
                nr.at[pl.ds(r * C, C)], sem))
            cps.append(pltpu.async_copy(
                aux_tab.at[ianeg.at[pl.ds(r * C, C)]],
                anr.at[pl.ds(r * C, C)], sem))
        return cps

    def compute(ck, bi):
        off = base + ck * C
        ur, vr, nr, aur, avr, anr = bufs[bi]
        dots_pos, dots_aux, dots_neg, dots_auxneg = dbufs[bi]

        def group(g, _):
            def samp(i, accs):
                apos, aaux, aneg, aaneg = accs
                s = g * L + i
                sel = lane == i
                urow = ur.at[s]
                vrow = vr.at[s]
                us = [urow[pl.ds(k * L, L)] for k in range(ND)]
                p = us[0] * vrow[pl.ds(DIM, L)]
                for k in range(1, ND):
                    p = p + us[k] * vrow[pl.ds(DIM + k * L, L)]
                apos = jnp.where(sel, _lane_sum(p, rots), apos)

                aurow = aur.at[s]
                avrow = avr.at[s]
                aus = [aurow[pl.ds(k * L, L)] for k in range(NAD)]
                a = aus[0] * avrow[pl.ds(AUX_DIM, L)]
                for k in range(1, NAD):
                    a = a + aus[k] * avrow[pl.ds(AUX_DIM + k * L, L)]
                aaux = jnp.where(sel, _lane_sum(a, rots), aaux)

                aneg2, aaneg2 = [], []
                for n in range(NNEG):
                    nrow = nr.at[n * C + s]
                    q = us[0] * nrow[pl.ds(DIM, L)]
                    for k in range(1, ND):
                        q = q + us[k] * nrow[pl.ds(DIM + k * L, L)]
                    aneg2.append(jnp.where(sel, _lane_sum(q, rots), aneg[n]))

                    anrow = anr.at[n * C + s]
                    aq = aus[0] * anrow[pl.ds(AUX_DIM, L)]
                    for k in range(1, NAD):
                        aq = aq + aus[k] * anrow[pl.ds(AUX_DIM + k * L, L)]
                    aaneg2.append(
                        jnp.where(sel, _lane_sum(aq, rots), aaneg[n]))
                return apos, aaux, tuple(aneg2), tuple(aaneg2)

            init = (zero, zero, (zero,) * NNEG, (zero,) * NNEG)
            apos, aaux, aneg, aaneg = lax.fori_loop(0, L, samp, init)

            dots_pos[pl.ds(g * L, L)] = apos
            dots_aux[pl.ds(g * L, L)] = aaux
            for n in range(NNEG):
                dots_neg[pl.ds(n * C + g * L, L)] = aneg[n]
                dots_auxneg[pl.ds(n * C + g * L, L)] = aaneg[n]
            return 0

        lax.fori_loop(0, NG, group, 0)

        sem = semd[bi]
        cps = [pltpu.async_copy(dots_pos, pos_out.at[pl.ds(off, C)], sem),
               pltpu.async_copy(dots_aux, aux_out.at[pl.ds(off, C)], sem)]
        for n in range(NNEG):
            cps.append(pltpu.async_copy(
                dots_neg.at[pl.ds(n * C, C)],
                neg_out.at[pl.ds(n * B + off, C)], sem))
            cps.append(pltpu.async_copy(
                dots_auxneg.at[pl.ds(n * C, C)],
                auxneg_out.at[pl.ds(n * B + off, C)], sem))
        return cps

    i_next = stage(0)
    for cp in i_next:
        cp.wait()
    g_cps = fire(0)
    i_next = stage(1) if NCHUNK > 1 else []
    d_prev = [[], []]
    for ck in range(NCHUNK):
        for cp in g_cps:
            cp.wait()
        if ck + 1 < NCHUNK:
            for cp in i_next:
                cp.wait()
            g_cps = fire(ck + 1)
            if ck + 2 < NCHUNK:
                i_next = stage(ck + 2)
        for cp in d_prev[ck % 2]:
            cp.wait()
        d_prev[ck % 2] = compute(ck, ck % 2)
    for par in (0, 1):
        for cp in d_prev[par]:
            cp.wait()


@functools.cache
def _make_sc_dots():
  return functools.partial(
    pl.kernel,
    out_type=[
        jax.ShapeDtypeStruct((B,), jnp.float32),
        jax.ShapeDtypeStruct((B,), jnp.float32),
        jax.ShapeDtypeStruct((NNEG * B,), jnp.float32),
        jax.ShapeDtypeStruct((NNEG * B,), jnp.float32),
    ],
    mesh=plsc.VectorSubcoreMesh(core_axis_name="c", subcore_axis_name="s",
                                num_cores=NC, num_subcores=NS),
    scratch_types=(
        [pltpu.VMEM((C,), jnp.int32),
         pltpu.VMEM((C,), jnp.int32),
         pltpu.VMEM((C * NNEG,), jnp.int32),
         pltpu.VMEM((C,), jnp.int32),
         pltpu.VMEM((C,), jnp.int32),
         pltpu.VMEM((C * NNEG,), jnp.int32)] * 2
        + [pltpu.VMEM((C, 2 * DIM), jnp.float32),
           pltpu.VMEM((C, 2 * DIM), jnp.float32),
           pltpu.VMEM((C * NNEG, 2 * DIM), jnp.float32),
           pltpu.VMEM((C, 4 * AUX_DIM), jnp.float32),
           pltpu.VMEM((C, 4 * AUX_DIM), jnp.float32),
           pltpu.VMEM((C * NNEG, 4 * AUX_DIM), jnp.float32)] * 2
        + [pltpu.VMEM((C,), jnp.float32),
           pltpu.VMEM((C,), jnp.float32),
           pltpu.VMEM((C * NNEG,), jnp.float32),
           pltpu.VMEM((C * NNEG,), jnp.float32)] * 2
        + [pltpu.SemaphoreType.DMA] * 6
    ),
  )(_sc_body)


def _pack_body(ut_ref, vt_ref, o_ref):
    d = ut_ref.shape[0]
    iod = lax.broadcasted_iota(jnp.int32, (d, 128), 0)
    ioc = lax.broadcasted_iota(jnp.int32, (d, 128), 1)
    e_lo = (ioc == iod).astype(jnp.float32)
    e_hi = (ioc == iod + d).astype(jnp.float32)
    dn = (((0,), (0,)), ((), ()))
    o_ref[...] = (
        lax.dot_general(ut_ref[...], e_lo, dn,
                        preferred_element_type=jnp.float32)
        + lax.dot_general(vt_ref[...], e_hi, dn,
                          preferred_element_type=jnp.float32))


def _pack_tables(ut, vt, w):
    d, v = ut.shape
    grid = (v + w - 1) // w
    return pl.pallas_call(
        _pack_body,
        grid=(grid,),
        in_specs=[pl.BlockSpec((d, w), lambda j: (0, j)),
                  pl.BlockSpec((d, w), lambda j: (0, j))],
        out_specs=pl.BlockSpec((w, 128), lambda j: (j, 0)),
        out_shape=jax.ShapeDtypeStruct((v, 128), jnp.float32),
        compiler_params=pltpu.CompilerParams(
            fuse_transposed_lhs_in_matmul=True),
    )(ut, vt)


def _tc_body(pos_ref, aux_ref, neg_ref, aneg_ref, o1_ref, o2_ref):
    pos = pos_ref[...]
    f1 = -jnp.log(jnp.clip(pos, EPS, 1.0 - EPS))
    neg = neg_ref[...]
    g1 = jnp.log(1.0 - jnp.clip(neg, EPS, 1.0 - EPS))
    o1_ref[...] = jnp.reshape((jnp.sum(f1) - jnp.sum(g1)) / B, (1, 1))

    aux = aux_ref[...]
    f2 = -jnp.log(jnp.clip(aux, EPS, 1.0 - EPS))
    an = aneg_ref[...]
    g2 = jnp.log(1.0 - jnp.clip(an, EPS, 1.0 - EPS))
    o2_ref[...] = jnp.reshape((jnp.sum(f2) - jnp.sum(g2)) / B, (1, 1))


def _tc_loss(pos_d, aux_d, neg_d, aneg_d):
    return pl.pallas_call(
        _tc_body,
        out_shape=[jax.ShapeDtypeStruct((1, 1), jnp.float32),
                   jax.ShapeDtypeStruct((1, 1), jnp.float32)],
    )(pos_d, aux_d, neg_d, aneg_d)


def kernel(pos_u, pos_v, neg_v, aux_pos_u, aux_pos_v, aux_neg_v,
           u_emb, v_emb, aux_u_emb, aux_v_emb):
    pos_u = pos_u.astype(jnp.int32)
    pos_v = pos_v.astype(jnp.int32)
    neg_flat = neg_v.reshape(-1).astype(jnp.int32)
    aux_pos_u = aux_pos_u.astype(jnp.int32)
    aux_pos_v = aux_pos_v.astype(jnp.int32)
    aneg_flat = aux_neg_v.reshape(-1).astype(jnp.int32)

    uv_tab = _pack_tables(u_emb.T, v_emb.T, 16384)
    aux_tab = _pack_tables(aux_u_emb.T, aux_v_emb.T, 16384)

    pos_d, aux_d, neg_d, aneg_d = _make_sc_dots()(
        pos_u, pos_v, neg_flat, aux_pos_u, aux_pos_v, aneg_flat,
        uv_tab, aux_tab)

    o1, o2 = _tc_loss(pos_d.reshape(B // 128, 128),
                      aux_d.reshape(B // 128, 128),
                      neg_d.reshape(NNEG * B // 128, 128),
                      aneg_d.reshape(NNEG * B // 128, 128))
    return (o1[0, 0], o2[0, 0])
